# baseline (speedup 1.0000x reference)
import jax
import jax.numpy as jnp
from jax import lax
from jax.experimental import pallas as pl
from jax.experimental.pallas import tpu as pltpu
from jax.experimental.pallas import tpu_sc as plsc

N_TOTAL = 50000
N_SRC = 10000
N_DST = 10000
E = 320000
IN_DIM = 128
OUT_DIM = 64
N_NOR = 25000

NP = 16000
ZROWS = 10112
NC = 2
NS = 16
NW = NC * NS
CHUNK = 128
ECH_W = 80
E_PAD = NW * CHUNK * ECH_W
NOR_CH_W = 8
NOR_PAD = NW * CHUNK * NOR_CH_W
OUT_CH_W = 8
OUT_PAD = NS * CHUNK * OUT_CH_W
NB = 4

R_BLK = 2000
N_GRID = N_DST // R_BLK
BETA = float(0.9 ** 5)


def _encoder_body(x_ref, w_ref, b_ref, o_ref):
    y = jnp.dot(x_ref[...], w_ref[...], preferred_element_type=jnp.float32)
    y = jnp.maximum(y + b_ref[...], 0.0)
    n = jnp.sqrt(jnp.sum(y * y, axis=1, keepdims=True))
    o_ref[...] = y / jnp.maximum(n, 1e-12)


def _sc_body(h_hbm, src2d, dst2d, nor2d, out2d, pdiff, labl, z_td, z_t,
             agg_out, cnt_out, pdn_out, pdo_out, lab_out,
             sidx_v, didx_v, rows_v, ones_v, gidx_v, gval_v, lval_v,
             acc_sh, cnt_sh, gsem, ssem, osem):
    c = lax.axis_index("c")
    s = lax.axis_index("s")
    wid = s * NC + c

    rps = ZROWS // NS
    r0 = s * rps
    pltpu.sync_copy(z_td.at[pl.ds(r0, rps)], acc_sh.at[pl.ds(r0, rps)])
    pltpu.sync_copy(z_t.at[pl.ds(r0, rps)], cnt_sh.at[pl.ds(r0, rps)])
    for i in range(CHUNK // 16):
        ones_v[pl.ds(i * 16, 16)] = jnp.full((16,), 1.0, jnp.float32)
    plsc.subcore_barrier()

    base = wid * ECH_W
    pltpu.sync_copy(src2d.at[pl.ds(base, ECH_W)], sidx_v)
    pltpu.sync_copy(dst2d.at[pl.ds(base, ECH_W)], didx_v)

    for b in range(NB):
        pltpu.async_copy(h_hbm.at[sidx_v.at[b]], rows_v.at[b], gsem.at[b])

    def edge_group(g, carry):
        for b in range(NB):
            j = g * NB + b
            pltpu.make_async_copy(h_hbm.at[sidx_v.at[j]], rows_v.at[b],
                                  gsem.at[b]).wait()
            pltpu.async_copy(rows_v.at[b], acc_sh.at[didx_v.at[j]],
                             ssem.at[b], add=True)
            pltpu.async_copy(ones_v, cnt_sh.at[didx_v.at[j]], osem, add=True)

            @pl.when(j >= NB)
            def _():
                pltpu.make_async_copy(z_t.at[pl.ds(0, CHUNK)],
                                      cnt_sh.at[pl.ds(0, CHUNK)],
                                      osem).wait()

            pb = (b - 1) % NB
            jn = j - 1 + NB

            @pl.when((j >= 1) & (jn < ECH_W))
            def _():
                pltpu.make_async_copy(rows_v.at[pb], acc_sh.at[didx_v.at[0]],
                                      ssem.at[pb]).wait()
                pltpu.async_copy(h_hbm.at[sidx_v.at[jn]], rows_v.at[pb],
                                 gsem.at[pb])

        return carry

    lax.fori_loop(0, ECH_W // NB, edge_group, 0)
    for b in range(NB):
        pltpu.make_async_copy(rows_v.at[b], acc_sh.at[didx_v.at[0]],
                              ssem.at[b]).wait()
    pltpu.make_async_copy(z_t.at[pl.ds(0, NB * CHUNK)],
                          cnt_sh.at[pl.ds(0, NB * CHUNK)], osem).wait()

    pltpu.sync_copy(nor2d.at[pl.ds(wid * NOR_CH_W, NOR_CH_W)], gidx_v)

    def nor_step(j, carry):
        pltpu.sync_copy(pdiff.at[gidx_v.at[j]], gval_v.at[j])
        return carry

    lax.fori_loop(0, NOR_CH_W, nor_step, 0)
    pltpu.sync_copy(gval_v, pdn_out.at[pl.ds(wid * NOR_CH_W, NOR_CH_W)])

    @pl.when(wid < NS)
    def _():
        pltpu.sync_copy(out2d.at[pl.ds(wid * OUT_CH_W, OUT_CH_W)],
                        gidx_v.at[pl.ds(0, OUT_CH_W)])

        def out_step(j, carry):
            pltpu.sync_copy(pdiff.at[gidx_v.at[j]], gval_v.at[j])
            pltpu.sync_copy(labl.at[gidx_v.at[j]], lval_v.at[j])
            return carry

        lax.fori_loop(0, OUT_CH_W, out_step, 0)
        pltpu.sync_copy(gval_v.at[pl.ds(0, OUT_CH_W)],
                        pdo_out.at[pl.ds(wid * OUT_CH_W, OUT_CH_W)])
        pltpu.sync_copy(lval_v.at[pl.ds(0, OUT_CH_W)],
                        lab_out.at[pl.ds(wid * OUT_CH_W, OUT_CH_W)])

    plsc.subcore_barrier()
    pltpu.sync_copy(acc_sh.at[pl.ds(r0, rps)],
                    agg_out.at[pl.ds(c * NP + r0, rps)])
    pltpu.sync_copy(cnt_sh.at[pl.ds(r0, rps)],
                    cnt_out.at[pl.ds(c * NP + r0, rps)])


def _final_body(h_ref, a0_ref, a1_ref, c0_ref, c1_ref, pdo_ref, lab_ref,
                pdn_ref, cen_ref, scores_ref, loss_ref, smem):
    i = pl.program_id(0)

    @pl.when(i == 0)
    def _():
        pdnv = pdn_ref[...]
        r = lax.broadcasted_iota(jnp.int32, pdnv.shape, 0)
        q = lax.broadcasted_iota(jnp.int32, pdnv.shape, 1)
        mask = (r * CHUNK + q) < N_NOR
        msum = jnp.sum(jnp.where(mask, pdnv, 0.0))
        mss = jnp.sum(jnp.where(mask, pdnv * pdnv, 0.0))
        n = jnp.float32(N_NOR)
        mean = msum / n
        var = (mss - msum * msum / n) / (n - 1.0)
        smem[4] = mean
        smem[5] = jnp.sqrt(var)
        smem[0] = 0.0
        smem[1] = 0.0
        smem[2] = 0.0
        smem[3] = 0.0

    mean = smem[4]
    std = smem[5]
    h = h_ref[...]
    mean_h = (a0_ref[...] + a1_ref[...]) / jnp.maximum(c0_ref[...] + c1_ref[...], 1.0)
    pdo = pdo_ref[...]
    pre = 1.0 - 1.0 / (1.0 + jnp.exp(-((pdo - mean) / std)))
    post = jnp.sum(h * mean_h, axis=1, keepdims=True)
    nei = (BETA * pre + (1.0 - BETA) * post) * 0.2
    h_out = nei * mean_h + (1.0 - nei) * h
    sc = jnp.sum(h_out * cen_ref[...], axis=1, keepdims=True)
    scores_ref[...] = sc

    curr = lab_ref[...] > 0.5
    posm = jnp.where(curr, 0.0, 1.0)
    negm = jnp.where(curr, 1.0, 0.0)
    sp = jnp.maximum(sc, 0.0) + jnp.log1p(jnp.exp(-jnp.abs(sc)))
    smem[0] += jnp.sum((sp - sc) * posm)
    smem[1] += jnp.sum(posm)
    smem[2] += jnp.sum(sp * negm)
    smem[3] += jnp.sum(negm)

    @pl.when(i == N_GRID - 1)
    def _():
        loss_ref[...] = jnp.reshape(smem[0] / smem[1] + smem[2] / smem[3], (1, 1))


def kernel(feats, edge_index, out_nodes, epoch, W, b, center, pos_diff, labels, nor_idx):
    del epoch
    f32 = jnp.float32

    src = jnp.pad(edge_index[0], (0, E_PAD - E)).reshape(E_PAD // CHUNK, CHUNK)
    pad_dst = N_DST + (jnp.arange(E_PAD - E, dtype=jnp.int32) % (ZROWS - N_DST))
    dst = jnp.concatenate([edge_index[1], pad_dst]).reshape(E_PAD // CHUNK, CHUNK)
    nor2d = jnp.pad(nor_idx, (0, NOR_PAD - N_NOR)).reshape(NOR_PAD // CHUNK, CHUNK)
    out2d = jnp.pad(out_nodes, (0, OUT_PAD - N_DST)).reshape(OUT_PAD // CHUNK, CHUNK)
    z_td = jnp.zeros((ZROWS, OUT_DIM), f32)
    z_t = jnp.zeros((ZROWS,), f32)

    h = pl.pallas_call(
        _encoder_body,
        grid=(N_GRID,),
        in_specs=[
            pl.BlockSpec((R_BLK, IN_DIM), lambda i: (i, 0)),
            pl.BlockSpec((IN_DIM, OUT_DIM), lambda i: (0, 0)),
            pl.BlockSpec((1, OUT_DIM), lambda i: (0, 0)),
        ],
        out_specs=pl.BlockSpec((R_BLK, OUT_DIM), lambda i: (i, 0)),
        out_shape=jax.ShapeDtypeStruct((N_SRC, OUT_DIM), f32),
    )(feats, W, b.reshape(1, OUT_DIM))

    mesh = plsc.VectorSubcoreMesh(core_axis_name="c", subcore_axis_name="s",
                                  num_cores=NC, num_subcores=NS)
    sc_call = pl.kernel(
        _sc_body,
        out_type=(
            jax.ShapeDtypeStruct((NC * NP, OUT_DIM), f32),
            jax.ShapeDtypeStruct((NC * NP,), f32),
            jax.ShapeDtypeStruct((NOR_PAD // CHUNK, CHUNK), f32),
            jax.ShapeDtypeStruct((OUT_PAD // CHUNK, CHUNK), f32),
            jax.ShapeDtypeStruct((OUT_PAD // CHUNK, CHUNK), jnp.int32),
        ),
        mesh=mesh,
        compiler_params=pltpu.CompilerParams(use_tc_tiling_on_sc=False),
        scratch_types=[
            pltpu.VMEM((ECH_W, CHUNK), jnp.int32),
            pltpu.VMEM((ECH_W, CHUNK), jnp.int32),
            pltpu.VMEM((NB, CHUNK, OUT_DIM), f32),
            pltpu.VMEM((CHUNK,), f32),
            pltpu.VMEM((NOR_CH_W, CHUNK), jnp.int32),
            pltpu.VMEM((NOR_CH_W, CHUNK), f32),
            pltpu.VMEM((OUT_CH_W, CHUNK), jnp.int32),
            pltpu.VMEM_SHARED((ZROWS, OUT_DIM), f32),
            pltpu.VMEM_SHARED((ZROWS,), f32),
            pltpu.SemaphoreType.DMA((NB,)),
            pltpu.SemaphoreType.DMA((NB,)),
            pltpu.SemaphoreType.DMA,
        ],
    )
    agg, cnt, pdn, pdo, lab = sc_call(h, src, dst, nor2d, out2d,
                                      pos_diff, labels, z_td, z_t)

    scores2d, loss = pl.pallas_call(
        _final_body,
        grid=(N_GRID,),
        in_specs=[
            pl.BlockSpec((R_BLK, OUT_DIM), lambda i: (i, 0)),
            pl.BlockSpec((R_BLK, OUT_DIM), lambda i: (i, 0)),
            pl.BlockSpec((R_BLK, OUT_DIM), lambda i: (NP // R_BLK + i, 0)),
            pl.BlockSpec((R_BLK, 1), lambda i: (i, 0)),
            pl.BlockSpec((R_BLK, 1), lambda i: (NP // R_BLK + i, 0)),
            pl.BlockSpec((R_BLK, 1), lambda i: (i, 0)),
            pl.BlockSpec((R_BLK, 1), lambda i: (i, 0)),
            pl.BlockSpec((NOR_PAD // CHUNK, CHUNK), lambda i: (0, 0)),
            pl.BlockSpec((1, OUT_DIM), lambda i: (0, 0)),
        ],
        out_specs=[
            pl.BlockSpec((R_BLK, 1), lambda i: (i, 0)),
            pl.BlockSpec((1, 1), lambda i: (0, 0)),
        ],
        out_shape=[
            jax.ShapeDtypeStruct((N_DST, 1), f32),
            jax.ShapeDtypeStruct((1, 1), f32),
        ],
        scratch_shapes=[pltpu.SMEM((8,), f32)],
    )(
        h,
        agg,
        agg,
        cnt.reshape(NC * NP, 1),
        cnt.reshape(NC * NP, 1),
        pdo.reshape(OUT_PAD)[:N_DST].reshape(N_DST, 1),
        lab.reshape(OUT_PAD)[:N_DST].reshape(N_DST, 1).astype(f32),
        pdn,
        center.reshape(1, OUT_DIM),
    )

    return (loss[0, 0], scores2d[:, 0])


# speedup vs baseline: 1.0008x; 1.0008x over previous
import jax
import jax.numpy as jnp
from jax import lax
from jax.experimental import pallas as pl
from jax.experimental.pallas import tpu as pltpu
from jax.experimental.pallas import tpu_sc as plsc

N_TOTAL = 50000
N_SRC = 10000
N_DST = 10000
E = 320000
IN_DIM = 128
OUT_DIM = 64
N_NOR = 25000

NP = 16000
ZROWS = 10112
NC = 2
NS = 16
NW = NC * NS
CHUNK = 128
ECH_W = 80
E_PAD = NW * CHUNK * ECH_W
NOR_CH_W = 8
NOR_PAD = NW * CHUNK * NOR_CH_W
OUT_CH_W = 8
OUT_PAD = NS * CHUNK * OUT_CH_W
NB = 4

R_BLK = 2000
N_GRID = N_DST // R_BLK
BETA = float(0.9 ** 5)


def _encoder_body(x_ref, w_ref, b_ref, o_ref):
    y = jnp.dot(x_ref[...], w_ref[...], preferred_element_type=jnp.float32)
    y = jnp.maximum(y + b_ref[...], 0.0)
    n = jnp.sqrt(jnp.sum(y * y, axis=1, keepdims=True))
    o_ref[...] = y / jnp.maximum(n, 1e-12)


def _sc_body(h_hbm, src2d, dst2d, nor2d, out2d, pdiff, labl, z_td, z_t,
             agg_out, cnt_out, pdn_out, pdo_out, lab_out,
             sidx_v, didx_v, rows_v, ones_v, gidx_v, gval_v, lval_v,
             acc_sh, cnt_sh, gsem, ssem, osem):
    c = lax.axis_index("c")
    s = lax.axis_index("s")
    wid = s * NC + c

    rps = ZROWS // NS
    r0 = s * rps
    pltpu.sync_copy(z_td.at[pl.ds(r0, rps)], acc_sh.at[pl.ds(r0, rps)])
    pltpu.sync_copy(z_t.at[pl.ds(r0, rps)], cnt_sh.at[pl.ds(r0, rps)])
    for i in range(CHUNK // 16):
        ones_v[pl.ds(i * 16, 16)] = jnp.full((16,), 1.0, jnp.float32)
    plsc.subcore_barrier()

    with jax.named_scope("stage"):
        base = wid * ECH_W
        pltpu.sync_copy(src2d.at[pl.ds(base, ECH_W)], sidx_v)
        pltpu.sync_copy(dst2d.at[pl.ds(base, ECH_W)], didx_v)

    for b in range(NB):
        pltpu.async_copy(h_hbm.at[sidx_v.at[b]], rows_v.at[b], gsem.at[b])

    def edge_group(g, carry):
        for b in range(NB):
            j = g * NB + b
            pltpu.make_async_copy(h_hbm.at[sidx_v.at[j]], rows_v.at[b],
                                  gsem.at[b]).wait()
            pltpu.async_copy(rows_v.at[b], acc_sh.at[didx_v.at[j]],
                             ssem.at[b], add=True)
            pltpu.async_copy(ones_v, cnt_sh.at[didx_v.at[j]], osem, add=True)

            @pl.when(j >= NB)
            def _():
                pltpu.make_async_copy(z_t.at[pl.ds(0, CHUNK)],
                                      cnt_sh.at[pl.ds(0, CHUNK)],
                                      osem).wait()

            pb = (b - 1) % NB
            jn = j - 1 + NB

            @pl.when((j >= 1) & (jn < ECH_W))
            def _():
                pltpu.make_async_copy(rows_v.at[pb], acc_sh.at[didx_v.at[0]],
                                      ssem.at[pb]).wait()
                pltpu.async_copy(h_hbm.at[sidx_v.at[jn]], rows_v.at[pb],
                                 gsem.at[pb])

        return carry

    with jax.named_scope("edges"):
        lax.fori_loop(0, ECH_W // NB, edge_group, 0)
    with jax.named_scope("tail"):
        for b in range(NB):
            pltpu.make_async_copy(rows_v.at[b], acc_sh.at[didx_v.at[0]],
                                  ssem.at[b]).wait()
        pltpu.make_async_copy(z_t.at[pl.ds(0, NB * CHUNK)],
                              cnt_sh.at[pl.ds(0, NB * CHUNK)], osem).wait()

    pltpu.sync_copy(nor2d.at[pl.ds(wid * NOR_CH_W, NOR_CH_W)], gidx_v)

    def nor_step(j, carry):
        pltpu.sync_copy(pdiff.at[gidx_v.at[j]], gval_v.at[j])
        return carry

    with jax.named_scope("nor"):
        lax.fori_loop(0, NOR_CH_W, nor_step, 0)
        pltpu.sync_copy(gval_v, pdn_out.at[pl.ds(wid * NOR_CH_W, NOR_CH_W)])

    @pl.when(wid < NS)
    def _():
        pltpu.sync_copy(out2d.at[pl.ds(wid * OUT_CH_W, OUT_CH_W)],
                        gidx_v.at[pl.ds(0, OUT_CH_W)])

        def out_step(j, carry):
            pltpu.sync_copy(pdiff.at[gidx_v.at[j]], gval_v.at[j])
            pltpu.sync_copy(labl.at[gidx_v.at[j]], lval_v.at[j])
            return carry

        lax.fori_loop(0, OUT_CH_W, out_step, 0)
        pltpu.sync_copy(gval_v.at[pl.ds(0, OUT_CH_W)],
                        pdo_out.at[pl.ds(wid * OUT_CH_W, OUT_CH_W)])
        pltpu.sync_copy(lval_v.at[pl.ds(0, OUT_CH_W)],
                        lab_out.at[pl.ds(wid * OUT_CH_W, OUT_CH_W)])

    with jax.named_scope("barrier2"):
        plsc.subcore_barrier()
    with jax.named_scope("publish"):
        pltpu.sync_copy(acc_sh.at[pl.ds(r0, rps)],
                        agg_out.at[pl.ds(c * NP + r0, rps)])
        pltpu.sync_copy(cnt_sh.at[pl.ds(r0, rps)],
                        cnt_out.at[pl.ds(c * NP + r0, rps)])


def _final_body(h_ref, a0_ref, a1_ref, c0_ref, c1_ref, pdo_ref, lab_ref,
                pdn_ref, cen_ref, scores_ref, loss_ref, smem):
    i = pl.program_id(0)

    @pl.when(i == 0)
    def _():
        pdnv = pdn_ref[...]
        r = lax.broadcasted_iota(jnp.int32, pdnv.shape, 0)
        q = lax.broadcasted_iota(jnp.int32, pdnv.shape, 1)
        mask = (r * CHUNK + q) < N_NOR
        msum = jnp.sum(jnp.where(mask, pdnv, 0.0))
        mss = jnp.sum(jnp.where(mask, pdnv * pdnv, 0.0))
        n = jnp.float32(N_NOR)
        mean = msum / n
        var = (mss - msum * msum / n) / (n - 1.0)
        smem[4] = mean
        smem[5] = jnp.sqrt(var)
        smem[0] = 0.0
        smem[1] = 0.0
        smem[2] = 0.0
        smem[3] = 0.0

    mean = smem[4]
    std = smem[5]
    h = h_ref[...]
    mean_h = (a0_ref[...] + a1_ref[...]) / jnp.maximum(c0_ref[...] + c1_ref[...], 1.0)
    pdo = pdo_ref[...]
    pre = 1.0 - 1.0 / (1.0 + jnp.exp(-((pdo - mean) / std)))
    post = jnp.sum(h * mean_h, axis=1, keepdims=True)
    nei = (BETA * pre + (1.0 - BETA) * post) * 0.2
    h_out = nei * mean_h + (1.0 - nei) * h
    sc = jnp.sum(h_out * cen_ref[...], axis=1, keepdims=True)
    scores_ref[...] = sc

    curr = lab_ref[...] > 0.5
    posm = jnp.where(curr, 0.0, 1.0)
    negm = jnp.where(curr, 1.0, 0.0)
    sp = jnp.maximum(sc, 0.0) + jnp.log1p(jnp.exp(-jnp.abs(sc)))
    smem[0] += jnp.sum((sp - sc) * posm)
    smem[1] += jnp.sum(posm)
    smem[2] += jnp.sum(sp * negm)
    smem[3] += jnp.sum(negm)

    @pl.when(i == N_GRID - 1)
    def _():
        loss_ref[...] = jnp.reshape(smem[0] / smem[1] + smem[2] / smem[3], (1, 1))


def kernel(feats, edge_index, out_nodes, epoch, W, b, center, pos_diff, labels, nor_idx):
    del epoch
    f32 = jnp.float32

    src = jnp.pad(edge_index[0], (0, E_PAD - E)).reshape(E_PAD // CHUNK, CHUNK)
    pad_dst = N_DST + (jnp.arange(E_PAD - E, dtype=jnp.int32) % (ZROWS - N_DST))
    dst = jnp.concatenate([edge_index[1], pad_dst]).reshape(E_PAD // CHUNK, CHUNK)
    nor2d = jnp.pad(nor_idx, (0, NOR_PAD - N_NOR)).reshape(NOR_PAD // CHUNK, CHUNK)
    out2d = jnp.pad(out_nodes, (0, OUT_PAD - N_DST)).reshape(OUT_PAD // CHUNK, CHUNK)
    z_td = jnp.zeros((ZROWS, OUT_DIM), f32)
    z_t = jnp.zeros((ZROWS,), f32)

    h = pl.pallas_call(
        _encoder_body,
        grid=(N_GRID,),
        in_specs=[
            pl.BlockSpec((R_BLK, IN_DIM), lambda i: (i, 0)),
            pl.BlockSpec((IN_DIM, OUT_DIM), lambda i: (0, 0)),
            pl.BlockSpec((1, OUT_DIM), lambda i: (0, 0)),
        ],
        out_specs=pl.BlockSpec((R_BLK, OUT_DIM), lambda i: (i, 0)),
        out_shape=jax.ShapeDtypeStruct((N_SRC, OUT_DIM), f32),
    )(feats, W, b.reshape(1, OUT_DIM))

    mesh = plsc.VectorSubcoreMesh(core_axis_name="c", subcore_axis_name="s",
                                  num_cores=NC, num_subcores=NS)
    sc_call = pl.kernel(
        _sc_body,
        out_type=(
            jax.ShapeDtypeStruct((NC * NP, OUT_DIM), f32),
            jax.ShapeDtypeStruct((NC * NP,), f32),
            jax.ShapeDtypeStruct((NOR_PAD // CHUNK, CHUNK), f32),
            jax.ShapeDtypeStruct((OUT_PAD // CHUNK, CHUNK), f32),
            jax.ShapeDtypeStruct((OUT_PAD // CHUNK, CHUNK), jnp.int32),
        ),
        mesh=mesh,
        compiler_params=pltpu.CompilerParams(use_tc_tiling_on_sc=False),
        scratch_types=[
            pltpu.VMEM((ECH_W, CHUNK), jnp.int32),
            pltpu.VMEM((ECH_W, CHUNK), jnp.int32),
            pltpu.VMEM((NB, CHUNK, OUT_DIM), f32),
            pltpu.VMEM((CHUNK,), f32),
            pltpu.VMEM((NOR_CH_W, CHUNK), jnp.int32),
            pltpu.VMEM((NOR_CH_W, CHUNK), f32),
            pltpu.VMEM((OUT_CH_W, CHUNK), jnp.int32),
            pltpu.VMEM_SHARED((ZROWS, OUT_DIM), f32),
            pltpu.VMEM_SHARED((ZROWS,), f32),
            pltpu.SemaphoreType.DMA((NB,)),
            pltpu.SemaphoreType.DMA((NB,)),
            pltpu.SemaphoreType.DMA,
        ],
    )
    agg, cnt, pdn, pdo, lab = sc_call(h, src, dst, nor2d, out2d,
                                      pos_diff, labels, z_td, z_t)

    scores2d, loss = pl.pallas_call(
        _final_body,
        grid=(N_GRID,),
        in_specs=[
            pl.BlockSpec((R_BLK, OUT_DIM), lambda i: (i, 0)),
            pl.BlockSpec((R_BLK, OUT_DIM), lambda i: (i, 0)),
            pl.BlockSpec((R_BLK, OUT_DIM), lambda i: (NP // R_BLK + i, 0)),
            pl.BlockSpec((R_BLK, 1), lambda i: (i, 0)),
            pl.BlockSpec((R_BLK, 1), lambda i: (NP // R_BLK + i, 0)),
            pl.BlockSpec((R_BLK, 1), lambda i: (i, 0)),
            pl.BlockSpec((R_BLK, 1), lambda i: (i, 0)),
            pl.BlockSpec((NOR_PAD // CHUNK, CHUNK), lambda i: (0, 0)),
            pl.BlockSpec((1, OUT_DIM), lambda i: (0, 0)),
        ],
        out_specs=[
            pl.BlockSpec((R_BLK, 1), lambda i: (i, 0)),
            pl.BlockSpec((1, 1), lambda i: (0, 0)),
        ],
        out_shape=[
            jax.ShapeDtypeStruct((N_DST, 1), f32),
            jax.ShapeDtypeStruct((1, 1), f32),
        ],
        scratch_shapes=[pltpu.SMEM((8,), f32)],
    )(
        h,
        agg,
        agg,
        cnt.reshape(NC * NP, 1),
        cnt.reshape(NC * NP, 1),
        pdo.reshape(OUT_PAD)[:N_DST].reshape(N_DST, 1),
        lab.reshape(OUT_PAD)[:N_DST].reshape(N_DST, 1).astype(f32),
        pdn,
        center.reshape(1, OUT_DIM),
    )

    return (loss[0, 0], scores2d[:, 0])


# speedup vs baseline: 2.0111x; 2.0095x over previous
import jax
import jax.numpy as jnp
from jax import lax
from jax.experimental import pallas as pl
from jax.experimental.pallas import tpu as pltpu
from jax.experimental.pallas import tpu_sc as plsc

N_TOTAL = 50000
N_SRC = 10000
N_DST = 10000
E = 320000
IN_DIM = 128
OUT_DIM = 64
N_NOR = 25000

NP = 16000
ZROWS = 10112
NC = 2
NS = 16
NW = NC * NS
CHUNK = 128
ECH_W = 80
E_PAD = NW * CHUNK * ECH_W
NOR_CH_W = 8
NOR_PAD = NW * CHUNK * NOR_CH_W
OUT_CH_W = 8
OUT_PAD = NS * CHUNK * OUT_CH_W
NB = 4

R_BLK = 2000
N_GRID = N_DST // R_BLK
BETA = float(0.9 ** 5)


def _encoder_body(x_ref, w_ref, b_ref, o_ref):
    y = jnp.dot(x_ref[...], w_ref[...], preferred_element_type=jnp.float32)
    y = jnp.maximum(y + b_ref[...], 0.0)
    n = jnp.sqrt(jnp.sum(y * y, axis=1, keepdims=True))
    o_ref[...] = y / jnp.maximum(n, 1e-12)


def _sc_body(h_hbm, src2d, dst2d, nor2d, out2d, pdiff, labl, z_td, z_t,
             agg_out, cnt_out, pdn_out, pdo_out, lab_out,
             sidx_v, didx_v, rows_v, ones_v, gidx_v, gval_v, lval_v,
             acc_sh, cnt_sh, gsem, ssem, osem):
    c = lax.axis_index("c")
    s = lax.axis_index("s")
    wid = s * NC + c

    rps = ZROWS // NS
    r0 = s * rps
    pltpu.sync_copy(z_td.at[pl.ds(r0, rps)], acc_sh.at[pl.ds(r0, rps)])
    pltpu.sync_copy(z_t.at[pl.ds(r0, rps)], cnt_sh.at[pl.ds(r0, rps)])
    for i in range(CHUNK // 16):
        ones_v[pl.ds(i * 16, 16)] = jnp.full((16,), 1.0, jnp.float32)
    plsc.subcore_barrier()

    with jax.named_scope("stage"):
        base = wid * ECH_W
        pltpu.sync_copy(src2d.at[pl.ds(base, ECH_W)], sidx_v)
        pltpu.sync_copy(dst2d.at[pl.ds(base, ECH_W)], didx_v)

    for b in range(NB):
        pltpu.async_copy(h_hbm.at[sidx_v.at[b]], rows_v.at[b], gsem.at[b])

    def edge_group(g, carry):
        for b in range(NB):
            j = g * NB + b
            pltpu.make_async_copy(h_hbm.at[sidx_v.at[j]], rows_v.at[b],
                                  gsem.at[b]).wait()
            pltpu.async_copy(rows_v.at[b], acc_sh.at[didx_v.at[j]],
                             ssem.at[b], add=True)
            pltpu.async_copy(ones_v, cnt_sh.at[didx_v.at[j]], osem, add=True)

            @pl.when(j >= NB)
            def _():
                pltpu.make_async_copy(z_t.at[pl.ds(0, CHUNK)],
                                      cnt_sh.at[pl.ds(0, CHUNK)],
                                      osem).wait()

            pb = (b - 1) % NB
            jn = j - 1 + NB

            @pl.when((j >= 1) & (jn < ECH_W))
            def _():
                pltpu.make_async_copy(rows_v.at[pb], acc_sh.at[didx_v.at[0]],
                                      ssem.at[pb]).wait()
                pltpu.async_copy(h_hbm.at[sidx_v.at[jn]], rows_v.at[pb],
                                 gsem.at[pb])

        return carry

    with jax.named_scope("edges"):
        lax.fori_loop(0, ECH_W // NB, edge_group, 0)
    with jax.named_scope("tail"):
        for b in range(NB):
            pltpu.make_async_copy(rows_v.at[b], acc_sh.at[didx_v.at[0]],
                                  ssem.at[b]).wait()
        pltpu.make_async_copy(z_t.at[pl.ds(0, NB * CHUNK)],
                              cnt_sh.at[pl.ds(0, NB * CHUNK)], osem).wait()

    pltpu.sync_copy(nor2d.at[pl.ds(wid * NOR_CH_W, NOR_CH_W)], gidx_v)

    def nor_step(j, carry):
        pltpu.sync_copy(pdiff.at[gidx_v.at[j]], gval_v.at[j])
        return carry

    with jax.named_scope("nor"):
        lax.fori_loop(0, NOR_CH_W, nor_step, 0)
        pltpu.sync_copy(gval_v, pdn_out.at[pl.ds(wid * NOR_CH_W, NOR_CH_W)])

    @pl.when(wid < NS)
    def _():
        pltpu.sync_copy(out2d.at[pl.ds(wid * OUT_CH_W, OUT_CH_W)],
                        gidx_v.at[pl.ds(0, OUT_CH_W)])

        def out_step(j, carry):
            pltpu.sync_copy(pdiff.at[gidx_v.at[j]], gval_v.at[j])
            pltpu.sync_copy(labl.at[gidx_v.at[j]], lval_v.at[j])
            return carry

        lax.fori_loop(0, OUT_CH_W, out_step, 0)
        pltpu.sync_copy(gval_v.at[pl.ds(0, OUT_CH_W)],
                        pdo_out.at[pl.ds(wid * OUT_CH_W, OUT_CH_W)])
        pltpu.sync_copy(lval_v.at[pl.ds(0, OUT_CH_W)],
                        lab_out.at[pl.ds(wid * OUT_CH_W, OUT_CH_W)])

    with jax.named_scope("barrier2"):
        plsc.subcore_barrier()
    with jax.named_scope("publish"):
        pltpu.sync_copy(acc_sh.at[pl.ds(r0, rps)],
                        agg_out.at[pl.ds(c * NP + r0, rps)])
        pltpu.sync_copy(cnt_sh.at[pl.ds(r0, rps)],
                        cnt_out.at[pl.ds(c * NP + r0, rps)])


def _final_body(h_ref, a0_ref, a1_ref, c0_ref, c1_ref, pdo_ref, lab_ref,
                pdn_ref, cen_ref, scores_ref, loss_ref, smem):
    i = pl.program_id(0)

    @pl.when(i == 0)
    def _():
        pdnv = pdn_ref[...]
        r = lax.broadcasted_iota(jnp.int32, pdnv.shape, 0)
        q = lax.broadcasted_iota(jnp.int32, pdnv.shape, 1)
        mask = (r * CHUNK + q) < N_NOR
        msum = jnp.sum(jnp.where(mask, pdnv, 0.0))
        mss = jnp.sum(jnp.where(mask, pdnv * pdnv, 0.0))
        n = jnp.float32(N_NOR)
        mean = msum / n
        var = (mss - msum * msum / n) / (n - 1.0)
        smem[4] = mean
        smem[5] = jnp.sqrt(var)
        smem[0] = 0.0
        smem[1] = 0.0
        smem[2] = 0.0
        smem[3] = 0.0

    mean = smem[4]
    std = smem[5]
    h = h_ref[...]
    mean_h = (a0_ref[...] + a1_ref[...]) / jnp.maximum(c0_ref[...] + c1_ref[...], 1.0)
    pdo = pdo_ref[...]
    pre = 1.0 - 1.0 / (1.0 + jnp.exp(-((pdo - mean) / std)))
    post = jnp.sum(h * mean_h, axis=1, keepdims=True)
    nei = (BETA * pre + (1.0 - BETA) * post) * 0.2
    h_out = nei * mean_h + (1.0 - nei) * h
    sc = jnp.sum(h_out * cen_ref[...], axis=1, keepdims=True)
    scores_ref[...] = sc

    curr = lab_ref[...] > 0.5
    posm = jnp.where(curr, 0.0, 1.0)
    negm = jnp.where(curr, 1.0, 0.0)
    sp = jnp.maximum(sc, 0.0) + jnp.log1p(jnp.exp(-jnp.abs(sc)))
    smem[0] += jnp.sum((sp - sc) * posm)
    smem[1] += jnp.sum(posm)
    smem[2] += jnp.sum(sp * negm)
    smem[3] += jnp.sum(negm)

    @pl.when(i == N_GRID - 1)
    def _():
        loss_ref[...] = jnp.reshape(smem[0] / smem[1] + smem[2] / smem[3], (1, 1))


def kernel(feats, edge_index, out_nodes, epoch, W, b, center, pos_diff, labels, nor_idx):
    del epoch
    f32 = jnp.float32

    pad_src = jnp.arange(E_PAD - E, dtype=jnp.int32) % N_SRC
    src = jnp.concatenate([edge_index[0], pad_src]).reshape(E_PAD // CHUNK, CHUNK)
    pad_dst = N_DST + (jnp.arange(E_PAD - E, dtype=jnp.int32) % (ZROWS - N_DST))
    dst = jnp.concatenate([edge_index[1], pad_dst]).reshape(E_PAD // CHUNK, CHUNK)
    pad_nor = jnp.arange(NOR_PAD - N_NOR, dtype=jnp.int32) % N_TOTAL
    nor2d = jnp.concatenate([nor_idx, pad_nor]).reshape(NOR_PAD // CHUNK, CHUNK)
    pad_out = jnp.arange(OUT_PAD - N_DST, dtype=jnp.int32) % N_TOTAL
    out2d = jnp.concatenate([out_nodes, pad_out]).reshape(OUT_PAD // CHUNK, CHUNK)
    z_td = jnp.zeros((ZROWS, OUT_DIM), f32)
    z_t = jnp.zeros((ZROWS,), f32)

    h = pl.pallas_call(
        _encoder_body,
        grid=(N_GRID,),
        in_specs=[
            pl.BlockSpec((R_BLK, IN_DIM), lambda i: (i, 0)),
            pl.BlockSpec((IN_DIM, OUT_DIM), lambda i: (0, 0)),
            pl.BlockSpec((1, OUT_DIM), lambda i: (0, 0)),
        ],
        out_specs=pl.BlockSpec((R_BLK, OUT_DIM), lambda i: (i, 0)),
        out_shape=jax.ShapeDtypeStruct((N_SRC, OUT_DIM), f32),
    )(feats, W, b.reshape(1, OUT_DIM))

    mesh = plsc.VectorSubcoreMesh(core_axis_name="c", subcore_axis_name="s",
                                  num_cores=NC, num_subcores=NS)
    sc_call = pl.kernel(
        _sc_body,
        out_type=(
            jax.ShapeDtypeStruct((NC * NP, OUT_DIM), f32),
            jax.ShapeDtypeStruct((NC * NP,), f32),
            jax.ShapeDtypeStruct((NOR_PAD // CHUNK, CHUNK), f32),
            jax.ShapeDtypeStruct((OUT_PAD // CHUNK, CHUNK), f32),
            jax.ShapeDtypeStruct((OUT_PAD // CHUNK, CHUNK), jnp.int32),
        ),
        mesh=mesh,
        compiler_params=pltpu.CompilerParams(use_tc_tiling_on_sc=False),
        scratch_types=[
            pltpu.VMEM((ECH_W, CHUNK), jnp.int32),
            pltpu.VMEM((ECH_W, CHUNK), jnp.int32),
            pltpu.VMEM((NB, CHUNK, OUT_DIM), f32),
            pltpu.VMEM((CHUNK,), f32),
            pltpu.VMEM((NOR_CH_W, CHUNK), jnp.int32),
            pltpu.VMEM((NOR_CH_W, CHUNK), f32),
            pltpu.VMEM((OUT_CH_W, CHUNK), jnp.int32),
            pltpu.VMEM_SHARED((ZROWS, OUT_DIM), f32),
            pltpu.VMEM_SHARED((ZROWS,), f32),
            pltpu.SemaphoreType.DMA((NB,)),
            pltpu.SemaphoreType.DMA((NB,)),
            pltpu.SemaphoreType.DMA,
        ],
    )
    agg, cnt, pdn, pdo, lab = sc_call(h, src, dst, nor2d, out2d,
                                      pos_diff, labels, z_td, z_t)

    scores2d, loss = pl.pallas_call(
        _final_body,
        grid=(N_GRID,),
        in_specs=[
            pl.BlockSpec((R_BLK, OUT_DIM), lambda i: (i, 0)),
            pl.BlockSpec((R_BLK, OUT_DIM), lambda i: (i, 0)),
            pl.BlockSpec((R_BLK, OUT_DIM), lambda i: (NP // R_BLK + i, 0)),
            pl.BlockSpec((R_BLK, 1), lambda i: (i, 0)),
            pl.BlockSpec((R_BLK, 1), lambda i: (NP // R_BLK + i, 0)),
            pl.BlockSpec((R_BLK, 1), lambda i: (i, 0)),
            pl.BlockSpec((R_BLK, 1), lambda i: (i, 0)),
            pl.BlockSpec((NOR_PAD // CHUNK, CHUNK), lambda i: (0, 0)),
            pl.BlockSpec((1, OUT_DIM), lambda i: (0, 0)),
        ],
        out_specs=[
            pl.BlockSpec((R_BLK, 1), lambda i: (i, 0)),
            pl.BlockSpec((1, 1), lambda i: (0, 0)),
        ],
        out_shape=[
            jax.ShapeDtypeStruct((N_DST, 1), f32),
            jax.ShapeDtypeStruct((1, 1), f32),
        ],
        scratch_shapes=[pltpu.SMEM((8,), f32)],
    )(
        h,
        agg,
        agg,
        cnt.reshape(NC * NP, 1),
        cnt.reshape(NC * NP, 1),
        pdo.reshape(OUT_PAD)[:N_DST].reshape(N_DST, 1),
        lab.reshape(OUT_PAD)[:N_DST].reshape(N_DST, 1).astype(f32),
        pdn,
        center.reshape(1, OUT_DIM),
    )

    return (loss[0, 0], scores2d[:, 0])


# speedup vs baseline: 2.0279x; 1.0083x over previous
import jax
import jax.numpy as jnp
from jax import lax
from jax.experimental import pallas as pl
from jax.experimental.pallas import tpu as pltpu
from jax.experimental.pallas import tpu_sc as plsc

N_TOTAL = 50000
N_SRC = 10000
N_DST = 10000
E = 320000
IN_DIM = 128
OUT_DIM = 64
N_NOR = 25000

NP = 16000
ZROWS = 10112
NC = 2
NS = 16
NW = NC * NS
CHUNK = 128
ECH_W = 80
E_PAD = NW * CHUNK * ECH_W
NOR_PAD = 32768
OUT_PAD = 16384
G_ROWS = (NOR_PAD + 2 * OUT_PAD) // CHUNK
G_CH_W = G_ROWS // NW
G_PD_W = NOR_PAD // CHUNK // G_CH_W + OUT_PAD // CHUNK // G_CH_W
NB = 4

R_BLK = 2000
N_GRID = N_DST // R_BLK
BETA = float(0.9 ** 5)


def _encoder_body(x_ref, w_ref, b_ref, o_ref):
    y = jnp.dot(x_ref[...], w_ref[...], preferred_element_type=jnp.float32)
    y = jnp.maximum(y + b_ref[...], 0.0)
    n = jnp.sqrt(jnp.sum(y * y, axis=1, keepdims=True))
    o_ref[...] = y / jnp.maximum(n, 1e-12)


def _sc_body(h_hbm, src2d, dst2d, gidx2d, pdiff_i, labl, z_td, z_t,
             agg_out, cnt_out, gv_out,
             sidx_v, didx_v, rows_v, ones_v, gidx_v, gval_v,
             acc_sh, cnt_sh, gsem, ssem, osem):
    c = lax.axis_index("c")
    s = lax.axis_index("s")
    wid = s * NC + c

    rps = ZROWS // NS
    r0 = s * rps
    pltpu.sync_copy(z_td.at[pl.ds(r0, rps)], acc_sh.at[pl.ds(r0, rps)])
    pltpu.sync_copy(z_t.at[pl.ds(r0, rps)], cnt_sh.at[pl.ds(r0, rps)])
    for i in range(CHUNK // 16):
        ones_v[pl.ds(i * 16, 16)] = jnp.full((16,), 1.0, jnp.float32)
    plsc.subcore_barrier()

    with jax.named_scope("stage"):
        base = wid * ECH_W
        pltpu.sync_copy(src2d.at[pl.ds(base, ECH_W)], sidx_v)
        pltpu.sync_copy(dst2d.at[pl.ds(base, ECH_W)], didx_v)

    for b in range(NB):
        pltpu.async_copy(h_hbm.at[sidx_v.at[b]], rows_v.at[b], gsem.at[b])

    def edge_group(g, carry):
        for b in range(NB):
            j = g * NB + b
            pltpu.make_async_copy(h_hbm.at[sidx_v.at[j]], rows_v.at[b],
                                  gsem.at[b]).wait()
            pltpu.async_copy(rows_v.at[b], acc_sh.at[didx_v.at[j]],
                             ssem.at[b], add=True)
            pltpu.async_copy(ones_v, cnt_sh.at[didx_v.at[j]], osem, add=True)

            @pl.when(j >= NB)
            def _():
                pltpu.make_async_copy(z_t.at[pl.ds(0, CHUNK)],
                                      cnt_sh.at[pl.ds(0, CHUNK)],
                                      osem).wait()

            pb = (b - 1) % NB
            jn = j - 1 + NB

            @pl.when((j >= 1) & (jn < ECH_W))
            def _():
                pltpu.make_async_copy(rows_v.at[pb], acc_sh.at[didx_v.at[0]],
                                      ssem.at[pb]).wait()
                pltpu.async_copy(h_hbm.at[sidx_v.at[jn]], rows_v.at[pb],
                                 gsem.at[pb])

        return carry

    with jax.named_scope("edges"):
        lax.fori_loop(0, ECH_W // NB, edge_group, 0)
    with jax.named_scope("tail"):
        for b in range(NB):
            pltpu.make_async_copy(rows_v.at[b], acc_sh.at[didx_v.at[0]],
                                  ssem.at[b]).wait()
        pltpu.make_async_copy(z_t.at[pl.ds(0, NB * CHUNK)],
                              cnt_sh.at[pl.ds(0, NB * CHUNK)], osem).wait()

    gbase = wid * G_CH_W
    pltpu.sync_copy(gidx2d.at[pl.ds(gbase, G_CH_W)], gidx_v)

    def pd_step(j, carry):
        pltpu.sync_copy(pdiff_i.at[gidx_v.at[j]], gval_v.at[j])
        return carry

    def lb_step(j, carry):
        pltpu.sync_copy(labl.at[gidx_v.at[j]], gval_v.at[j])
        return carry

    @pl.when(wid < G_PD_W)
    def _():
        lax.fori_loop(0, G_CH_W, pd_step, 0)

    @pl.when(wid >= G_PD_W)
    def _():
        lax.fori_loop(0, G_CH_W, lb_step, 0)

    pltpu.sync_copy(gval_v, gv_out.at[pl.ds(gbase, G_CH_W)])

    with jax.named_scope("barrier2"):
        plsc.subcore_barrier()
    with jax.named_scope("publish"):
        pltpu.sync_copy(acc_sh.at[pl.ds(r0, rps)],
                        agg_out.at[pl.ds(c * NP + r0, rps)])
        pltpu.sync_copy(cnt_sh.at[pl.ds(r0, rps)],
                        cnt_out.at[pl.ds(c * NP + r0, rps)])


def _final_body(h_ref, a0_ref, a1_ref, c0_ref, c1_ref, pdo_ref, lab_ref,
                pdn_ref, cen_ref, scores_ref, loss_ref, smem):
    i = pl.program_id(0)

    @pl.when(i == 0)
    def _():
        pdnv = pdn_ref[...]
        r = lax.broadcasted_iota(jnp.int32, pdnv.shape, 0)
        q = lax.broadcasted_iota(jnp.int32, pdnv.shape, 1)
        mask = (r * CHUNK + q) < N_NOR
        msum = jnp.sum(jnp.where(mask, pdnv, 0.0))
        mss = jnp.sum(jnp.where(mask, pdnv * pdnv, 0.0))
        n = jnp.float32(N_NOR)
        mean = msum / n
        var = (mss - msum * msum / n) / (n - 1.0)
        smem[4] = mean
        smem[5] = jnp.sqrt(var)
        smem[0] = 0.0
        smem[1] = 0.0
        smem[2] = 0.0
        smem[3] = 0.0

    mean = smem[4]
    std = smem[5]
    h = h_ref[...]
    mean_h = (a0_ref[...] + a1_ref[...]) / jnp.maximum(c0_ref[...] + c1_ref[...], 1.0)
    pdo = pdo_ref[...]
    pre = 1.0 - 1.0 / (1.0 + jnp.exp(-((pdo - mean) / std)))
    post = jnp.sum(h * mean_h, axis=1, keepdims=True)
    nei = (BETA * pre + (1.0 - BETA) * post) * 0.2
    h_out = nei * mean_h + (1.0 - nei) * h
    sc = jnp.sum(h_out * cen_ref[...], axis=1, keepdims=True)
    scores_ref[...] = sc

    curr = lab_ref[...] > 0.5
    posm = jnp.where(curr, 0.0, 1.0)
    negm = jnp.where(curr, 1.0, 0.0)
    sp = jnp.maximum(sc, 0.0) + jnp.log1p(jnp.exp(-jnp.abs(sc)))
    smem[0] += jnp.sum((sp - sc) * posm)
    smem[1] += jnp.sum(posm)
    smem[2] += jnp.sum(sp * negm)
    smem[3] += jnp.sum(negm)

    @pl.when(i == N_GRID - 1)
    def _():
        loss_ref[...] = jnp.reshape(smem[0] / smem[1] + smem[2] / smem[3], (1, 1))


def kernel(feats, edge_index, out_nodes, epoch, W, b, center, pos_diff, labels, nor_idx):
    del epoch
    f32 = jnp.float32

    pad_src = jnp.arange(E_PAD - E, dtype=jnp.int32) % N_SRC
    src = jnp.concatenate([edge_index[0], pad_src]).reshape(E_PAD // CHUNK, CHUNK)
    pad_dst = N_DST + (jnp.arange(E_PAD - E, dtype=jnp.int32) % (ZROWS - N_DST))
    dst = jnp.concatenate([edge_index[1], pad_dst]).reshape(E_PAD // CHUNK, CHUNK)
    pad_nor = jnp.arange(NOR_PAD - N_NOR, dtype=jnp.int32) % N_TOTAL
    pad_out = jnp.arange(OUT_PAD - N_DST, dtype=jnp.int32) % N_TOTAL
    outp = jnp.concatenate([out_nodes, pad_out])
    gidx2d = jnp.concatenate([nor_idx, pad_nor, outp, outp]).reshape(G_ROWS, CHUNK)
    z_td = jnp.zeros((ZROWS, OUT_DIM), f32)
    z_t = jnp.zeros((ZROWS,), f32)

    h = pl.pallas_call(
        _encoder_body,
        grid=(N_GRID,),
        in_specs=[
            pl.BlockSpec((R_BLK, IN_DIM), lambda i: (i, 0)),
            pl.BlockSpec((IN_DIM, OUT_DIM), lambda i: (0, 0)),
            pl.BlockSpec((1, OUT_DIM), lambda i: (0, 0)),
        ],
        out_specs=pl.BlockSpec((R_BLK, OUT_DIM), lambda i: (i, 0)),
        out_shape=jax.ShapeDtypeStruct((N_SRC, OUT_DIM), f32),
    )(feats, W, b.reshape(1, OUT_DIM))

    mesh = plsc.VectorSubcoreMesh(core_axis_name="c", subcore_axis_name="s",
                                  num_cores=NC, num_subcores=NS)
    sc_call = pl.kernel(
        _sc_body,
        out_type=(
            jax.ShapeDtypeStruct((NC * NP, OUT_DIM), f32),
            jax.ShapeDtypeStruct((NC * NP,), f32),
            jax.ShapeDtypeStruct((G_ROWS, CHUNK), jnp.int32),
        ),
        mesh=mesh,
        compiler_params=pltpu.CompilerParams(use_tc_tiling_on_sc=False),
        scratch_types=[
            pltpu.VMEM((ECH_W, CHUNK), jnp.int32),
            pltpu.VMEM((ECH_W, CHUNK), jnp.int32),
            pltpu.VMEM((NB, CHUNK, OUT_DIM), f32),
            pltpu.VMEM((CHUNK,), f32),
            pltpu.VMEM((G_CH_W, CHUNK), jnp.int32),
            pltpu.VMEM((G_CH_W, CHUNK), jnp.int32),
            pltpu.VMEM_SHARED((ZROWS, OUT_DIM), f32),
            pltpu.VMEM_SHARED((ZROWS,), f32),
            pltpu.SemaphoreType.DMA((NB,)),
            pltpu.SemaphoreType.DMA((NB,)),
            pltpu.SemaphoreType.DMA,
        ],
    )
    pos_diff_i = jax.lax.bitcast_convert_type(pos_diff, jnp.int32)
    agg, cnt, gv = sc_call(h, src, dst, gidx2d, pos_diff_i, labels, z_td, z_t)
    gvf = gv.reshape(-1)
    pdn = jax.lax.bitcast_convert_type(gvf[:NOR_PAD], f32).reshape(NOR_PAD // CHUNK, CHUNK)
    pdo = jax.lax.bitcast_convert_type(gvf[NOR_PAD:NOR_PAD + N_DST], f32)
    lab = gvf[NOR_PAD + OUT_PAD:NOR_PAD + OUT_PAD + N_DST]

    cntr = cnt.reshape(NC * NP, 1)
    scores2d, loss = pl.pallas_call(
        _final_body,
        grid=(N_GRID,),
        in_specs=[
            pl.BlockSpec((R_BLK, OUT_DIM), lambda i: (i, 0)),
            pl.BlockSpec((R_BLK, OUT_DIM), lambda i: (i, 0)),
            pl.BlockSpec((R_BLK, OUT_DIM), lambda i: (NP // R_BLK + i, 0)),
            pl.BlockSpec((R_BLK, 1), lambda i: (i, 0)),
            pl.BlockSpec((R_BLK, 1), lambda i: (NP // R_BLK + i, 0)),
            pl.BlockSpec((R_BLK, 1), lambda i: (i, 0)),
            pl.BlockSpec((R_BLK, 1), lambda i: (i, 0)),
            pl.BlockSpec((NOR_PAD // CHUNK, CHUNK), lambda i: (0, 0)),
            pl.BlockSpec((1, OUT_DIM), lambda i: (0, 0)),
        ],
        out_specs=[
            pl.BlockSpec((R_BLK, 1), lambda i: (i, 0)),
            pl.BlockSpec((1, 1), lambda i: (0, 0)),
        ],
        out_shape=[
            jax.ShapeDtypeStruct((N_DST, 1), f32),
            jax.ShapeDtypeStruct((1, 1), f32),
        ],
        scratch_shapes=[pltpu.SMEM((8,), f32)],
    )(
        h,
        agg,
        agg,
        cntr,
        cntr,
        pdo.reshape(N_DST, 1),
        lab.reshape(N_DST, 1).astype(f32),
        pdn,
        center.reshape(1, OUT_DIM),
    )

    return (loss[0, 0], scores2d[:, 0])
